# baseline (speedup 1.0000x reference)
import functools

import jax
import jax.numpy as jnp
from jax import lax
from jax.experimental import pallas as pl
from jax.experimental.pallas import tpu as pltpu
from jax.experimental.pallas import tpu_sc as plsc

_N = 10000
_C = 128
_E = 160000
_NS = 16
_RPT = 624
_TAIL = _N - _NS * _RPT
_CH = 128
_NCHUNK = 160
_EPT = 2 * _E // _NS
_NPAD = _NCHUNK * _CH - _EPT
_NA = _N + 8
_BM = 1000


def _mm_msg(h, w4, relu_in):
    nb = _N // _BM

    def body(h_ref, w_ref, o_ref):
        x = h_ref[...]
        if relu_in:
            x = jnp.maximum(x, 0.0)
        o_ref[...] = jnp.dot(x, w_ref[0], preferred_element_type=jnp.float32,
                             precision=lax.Precision.HIGHEST)

    return pl.pallas_call(
        body,
        grid=(4 * nb,),
        in_specs=[
            pl.BlockSpec((_BM, _C), lambda i: (((i // nb) % 2) * nb + i % nb, 0)),
            pl.BlockSpec((1, _C, _C), lambda i: (i // nb, 0, 0)),
        ],
        out_specs=pl.BlockSpec((_BM, _C), lambda i: (i, 0)),
        out_shape=jax.ShapeDtypeStruct((4 * _N, _C), jnp.float32),
    )(h, w4)


def _mm_init(h, w2, resid, relu_in):
    nb = _N // _BM

    def body(*refs):
        if resid is not None:
            h_ref, w_ref, r_ref, o_ref = refs
        else:
            h_ref, w_ref, o_ref = refs
        x = h_ref[...]
        if relu_in:
            x = jnp.maximum(x, 0.0)
        acc = jnp.dot(x, w_ref[0], preferred_element_type=jnp.float32,
                      precision=lax.Precision.HIGHEST)
        if resid is not None:
            acc = acc + r_ref[...]
        o_ref[...] = acc

    in_specs = [
        pl.BlockSpec((_BM, _C), lambda i: (i, 0)),
        pl.BlockSpec((1, _C, _C), lambda i: (i // nb, 0, 0)),
    ]
    args = [h, w2]
    if resid is not None:
        in_specs.append(pl.BlockSpec((_BM, _C), lambda i: (i, 0)))
        args.append(resid)
    return pl.pallas_call(
        body,
        grid=(2 * nb,),
        in_specs=in_specs,
        out_specs=pl.BlockSpec((_BM, _C), lambda i: (i, 0)),
        out_shape=jax.ShapeDtypeStruct((2 * _N, _C), jnp.float32),
    )(*args)


def _relu_k(z):
    def body(z_ref, o_ref):
        o_ref[...] = jnp.maximum(z_ref[...], 0.0)

    nb = 2 * _N // _BM
    return pl.pallas_call(
        body,
        grid=(nb,),
        in_specs=[pl.BlockSpec((_BM, _C), lambda i: (i, 0))],
        out_specs=pl.BlockSpec((_BM, _C), lambda i: (i, 0)),
        out_shape=jax.ShapeDtypeStruct((2 * _N, _C), jnp.float32),
    )(z)


def _sc_scatter(tables, init, idx):
    tables, init, idx = lax.optimization_barrier((tables, init, idx))
    mesh = plsc.VectorSubcoreMesh(core_axis_name="c", subcore_axis_name="s")

    @functools.partial(
        pl.kernel,
        out_type=jax.ShapeDtypeStruct((2 * _N, _C), jnp.float32),
        mesh=mesh,
        scratch_types=[
            pltpu.VMEM((4, 2, _CH), jnp.int32),
            pltpu.VMEM((2, _CH, _C), jnp.float32),
            pltpu.VMEM_SHARED((_NA, _C), jnp.float32),
            pltpu.SemaphoreType.DMA,
            pltpu.SemaphoreType.DMA,
            pltpu.SemaphoreType.DMA,
            pltpu.SemaphoreType.DMA,
            pltpu.SemaphoreType.DMA,
            pltpu.SemaphoreType.DMA,
            pltpu.SemaphoreType.DMA,
            pltpu.SemaphoreType.DMA,
        ],
    )
    def body(tab_hbm, init_hbm, idx_hbm, out_hbm, idx_v, rows_v, acc,
             gsem0, gsem1, ssem0, ssem1, isem0, isem1, isem2, isem3):
        gsem = (gsem0, gsem1)
        ssem = (ssem0, ssem1)
        isem = (isem0, isem1, isem2, isem3)
        c = lax.axis_index("c")
        s = lax.axis_index("s")
        r0 = c * _N + s * _RPT
        pltpu.sync_copy(init_hbm.at[pl.ds(r0, _RPT), :],
                        acc.at[pl.ds(s * _RPT, _RPT), :])

        @pl.when(s == _NS - 1)
        def _():
            pltpu.sync_copy(init_hbm.at[pl.ds(c * _N + _NS * _RPT, _TAIL), :],
                            acc.at[pl.ds(_NS * _RPT, _TAIL), :])

        plsc.subcore_barrier()

        def gather_half(q, p, h):
            return pltpu.make_async_copy(
                tab_hbm.at[idx_v.at[q, 0, pl.ds(h * 32, 32)]],
                rows_v.at[p, pl.ds(h * 32, 32), :], gsem[p])

        class gather_desc:
            def __init__(self, q, p):
                self.q, self.p = q, p
            def start(self):
                for h in range(4):
                    gather_half(self.q, self.p, h).start()
            def wait(self):
                for h in range(4):
                    gather_half(self.q, self.p, h).wait()

        def scatter_desc(q, p):
            return pltpu.make_async_copy(rows_v.at[p],
                                         acc.at[idx_v.at[q, 1]], ssem[p])

        def idx_desc(j, q):
            return pltpu.make_async_copy(idx_hbm.at[c, s, j],
                                         idx_v.at[q], isem[q])

        def load_idx(j, q):
            pltpu.async_copy(idx_hbm.at[c, s, j], idx_v.at[q], isem[q])

        def scatter(q, p):
            pltpu.async_copy(rows_v.at[p], acc.at[idx_v.at[q, 1]], ssem[p],
                             add=True)

        load_idx(0, 0)
        load_idx(1, 1)
        idx_desc(0, 0).wait()
        gather_desc(0, 0).start()

        def outer(g, carry):
            for b in range(4):
                j = 4 * g + b
                p = b % 2
                gather_desc(b, p).wait()
                if b == 0:
                    @pl.when(g >= 1)
                    def _():
                        scatter_desc((b + 3) % 4, 1 - p).wait()
                else:
                    scatter_desc((b + 3) % 4, 1 - p).wait()
                if b in (0, 1, 2):
                    idx_desc(j + 1, (b + 1) % 4).wait()
                    gather_desc((b + 1) % 4, 1 - p).start()
                else:
                    @pl.when(g <= 38)
                    def _():
                        idx_desc(j + 1, 0).wait()
                        gather_desc(0, 1 - p).start()
                scatter(b, p)
                if b in (0, 1):
                    load_idx(j + 2, (b + 2) % 4)
                else:
                    @pl.when(g <= 38)
                    def _():
                        load_idx(j + 2, (b + 2) % 4)
            return carry

        lax.fori_loop(0, _NCHUNK // 4, outer, 0)
        scatter_desc(3, 1).wait()
        plsc.subcore_barrier()
        pltpu.sync_copy(acc.at[pl.ds(s * _RPT, _RPT), :],
                        out_hbm.at[pl.ds(r0, _RPT), :])

        @pl.when(s == _NS - 1)
        def _():
            pltpu.sync_copy(acc.at[pl.ds(_NS * _RPT, _TAIL), :],
                            out_hbm.at[pl.ds(c * _N + _NS * _RPT, _TAIL), :])

    return body(tables, init, idx)


def _build_idx(dd, dp, pp):
    c0_src = jnp.stack([dd[0], dp[1] + _N])
    c0_dst = jnp.stack([dd[1], dp[0]])
    c1_src = jnp.stack([dp[0] + 2 * _N, pp[0] + 3 * _N])
    c1_dst = jnp.stack([dp[1], pp[1]])

    ept = _E // _NS
    pad_src = jnp.tile((jnp.arange(_NPAD, dtype=jnp.int32) * 83) % (4 * _N), (_NS, 1))
    pad_dst = jnp.full((_NS, _NPAD), _N, jnp.int32) + (
        jnp.arange(_NPAD, dtype=jnp.int32) % 8)[None, :]

    def lay(a, pad):
        a = a.reshape(2, _NS, ept).transpose(1, 0, 2).reshape(_NS, _EPT)
        a = jnp.concatenate([a, pad], axis=1)
        return a.reshape(_NS, _NCHUNK, _CH)

    def core(src2, dst2):
        return jnp.stack([lay(src2, pad_src), lay(dst2, pad_dst)], axis=2)

    return jnp.stack([core(c0_src, c0_dst), core(c1_src, c1_dst)])


def kernel(h_drug, h_prot, dd_edge_index, dp_edge_index, pp_edge_index,
           W1_ds, W1_ps, W1_dd, W1_pd, W1_dp, W1_pp,
           W2_ds, W2_ps, W2_dd, W2_pd, W2_dp, W2_pp):
    h = jnp.concatenate([h_drug, h_prot], axis=0)
    idx = _build_idx(dd_edge_index, dp_edge_index, pp_edge_index)

    wmsg1 = jnp.stack([W1_dd, W1_pd, W1_dp, W1_pp])
    winit1 = jnp.stack([W1_ds, W1_ps])
    wmsg2 = jnp.stack([W2_dd, W2_pd, W2_dp, W2_pp])
    winit2 = jnp.stack([W2_ds, W2_ps])

    t1 = _mm_msg(h, wmsg1, relu_in=False)
    i1 = _mm_init(h, winit1, resid=None, relu_in=False)
    z1 = _sc_scatter(t1, i1, idx)
    t2 = _mm_msg(z1, wmsg2, relu_in=True)
    i2 = _mm_init(z1, winit2, resid=h, relu_in=True)
    z2 = _sc_scatter(t2, i2, idx)
    out = _relu_k(z2)
    return out[:_N], out[_N:]


# speedup vs baseline: 1.0036x; 1.0036x over previous
import functools

import jax
import jax.numpy as jnp
from jax import lax
from jax.experimental import pallas as pl
from jax.experimental.pallas import tpu as pltpu
from jax.experimental.pallas import tpu_sc as plsc

_N = 10000
_C = 128
_E = 160000
_NS = 16
_RPT = 624
_TAIL = _N - _NS * _RPT
_CH = 128
_NCHUNK = 160
_EPT = 2 * _E // _NS
_NPAD = _NCHUNK * _CH - _EPT
_NA = _N + 8
_BM = 1000


def _mm_msg(h, w4, relu_in):
    nb = _N // _BM

    def body(h_ref, w_ref, o_ref):
        x = h_ref[...]
        if relu_in:
            x = jnp.maximum(x, 0.0)
        o_ref[...] = jnp.dot(x, w_ref[0], preferred_element_type=jnp.float32,
                             precision=lax.Precision.HIGHEST)

    return pl.pallas_call(
        body,
        grid=(4 * nb,),
        in_specs=[
            pl.BlockSpec((_BM, _C), lambda i: (((i // nb) % 2) * nb + i % nb, 0)),
            pl.BlockSpec((1, _C, _C), lambda i: (i // nb, 0, 0)),
        ],
        out_specs=pl.BlockSpec((_BM, _C), lambda i: (i, 0)),
        out_shape=jax.ShapeDtypeStruct((4 * _N, _C), jnp.float32),
    )(h, w4)


def _mm_init(h, w2, resid, relu_in):
    nb = _N // _BM

    def body(*refs):
        if resid is not None:
            h_ref, w_ref, r_ref, o_ref = refs
        else:
            h_ref, w_ref, o_ref = refs
        x = h_ref[...]
        if relu_in:
            x = jnp.maximum(x, 0.0)
        acc = jnp.dot(x, w_ref[0], preferred_element_type=jnp.float32,
                      precision=lax.Precision.HIGHEST)
        if resid is not None:
            acc = acc + r_ref[...]
        o_ref[...] = acc

    in_specs = [
        pl.BlockSpec((_BM, _C), lambda i: (i, 0)),
        pl.BlockSpec((1, _C, _C), lambda i: (i // nb, 0, 0)),
    ]
    args = [h, w2]
    if resid is not None:
        in_specs.append(pl.BlockSpec((_BM, _C), lambda i: (i, 0)))
        args.append(resid)
    return pl.pallas_call(
        body,
        grid=(2 * nb,),
        in_specs=in_specs,
        out_specs=pl.BlockSpec((_BM, _C), lambda i: (i, 0)),
        out_shape=jax.ShapeDtypeStruct((2 * _N, _C), jnp.float32),
    )(*args)


def _relu_k(z):
    def body(z_ref, o_ref):
        o_ref[...] = jnp.maximum(z_ref[...], 0.0)

    nb = 2 * _N // _BM
    return pl.pallas_call(
        body,
        grid=(nb,),
        in_specs=[pl.BlockSpec((_BM, _C), lambda i: (i, 0))],
        out_specs=pl.BlockSpec((_BM, _C), lambda i: (i, 0)),
        out_shape=jax.ShapeDtypeStruct((2 * _N, _C), jnp.float32),
    )(z)


def _sc_scatter(tables, init, idx):
    tables, init, idx = lax.optimization_barrier((tables, init, idx))
    mesh = plsc.VectorSubcoreMesh(core_axis_name="c", subcore_axis_name="s")

    @functools.partial(
        pl.kernel,
        out_type=jax.ShapeDtypeStruct((2 * _N, _C), jnp.float32),
        mesh=mesh,
        scratch_types=[
            pltpu.VMEM((4, 2, _CH), jnp.int32),
            pltpu.VMEM((2, _CH, _C), jnp.float32),
            pltpu.VMEM_SHARED((_NA, _C), jnp.float32),
            pltpu.SemaphoreType.DMA,
            pltpu.SemaphoreType.DMA,
            pltpu.SemaphoreType.DMA,
            pltpu.SemaphoreType.DMA,
            pltpu.SemaphoreType.DMA,
            pltpu.SemaphoreType.DMA,
            pltpu.SemaphoreType.DMA,
            pltpu.SemaphoreType.DMA,
        ],
    )
    def body(tab_hbm, init_hbm, idx_hbm, out_hbm, idx_v, rows_v, acc,
             gsem0, gsem1, ssem0, ssem1, isem0, isem1, isem2, isem3):
        gsem = (gsem0, gsem1)
        ssem = (ssem0, ssem1)
        isem = (isem0, isem1, isem2, isem3)
        c = lax.axis_index("c")
        s = lax.axis_index("s")
        r0 = c * _N + s * _RPT
        pltpu.sync_copy(init_hbm.at[pl.ds(r0, _RPT), :],
                        acc.at[pl.ds(s * _RPT, _RPT), :])

        @pl.when(s == _NS - 1)
        def _():
            pltpu.sync_copy(init_hbm.at[pl.ds(c * _N + _NS * _RPT, _TAIL), :],
                            acc.at[pl.ds(_NS * _RPT, _TAIL), :])

        plsc.subcore_barrier()

        def gather_desc(q, p):
            return pltpu.make_async_copy(tab_hbm.at[idx_v.at[q, 0]],
                                         rows_v.at[p], gsem[p])

        def scatter_desc(q, p):
            return pltpu.make_async_copy(rows_v.at[p],
                                         acc.at[idx_v.at[q, 1]], ssem[p])

        def idx_desc(j, q):
            return pltpu.make_async_copy(idx_hbm.at[c, s, j],
                                         idx_v.at[q], isem[q])

        def load_idx(j, q):
            pltpu.async_copy(idx_hbm.at[c, s, j], idx_v.at[q], isem[q])

        def scatter(q, p):
            pltpu.async_copy(rows_v.at[p], acc.at[idx_v.at[q, 1]], ssem[p],
                             add=True)

        load_idx(0, 0)
        load_idx(1, 1)
        idx_desc(0, 0).wait()
        gather_desc(0, 0).start()

        def outer(g, carry):
            for b in range(4):
                j = 4 * g + b
                p = b % 2
                gather_desc(b, p).wait()
                if b == 0:
                    @pl.when(g >= 1)
                    def _():
                        scatter_desc((b + 3) % 4, 1 - p).wait()
                else:
                    scatter_desc((b + 3) % 4, 1 - p).wait()
                if b in (0, 1, 2):
                    idx_desc(j + 1, (b + 1) % 4).wait()
                    gather_desc((b + 1) % 4, 1 - p).start()
                else:
                    @pl.when(g <= 38)
                    def _():
                        idx_desc(j + 1, 0).wait()
                        gather_desc(0, 1 - p).start()
                scatter(b, p)
                if b in (0, 1):
                    load_idx(j + 2, (b + 2) % 4)
                else:
                    @pl.when(g <= 38)
                    def _():
                        load_idx(j + 2, (b + 2) % 4)
            return carry

        lax.fori_loop(0, _NCHUNK // 4, outer, 0)
        scatter_desc(3, 1).wait()
        plsc.subcore_barrier()
        pltpu.sync_copy(acc.at[pl.ds(s * _RPT, _RPT), :],
                        out_hbm.at[pl.ds(r0, _RPT), :])

        @pl.when(s == _NS - 1)
        def _():
            pltpu.sync_copy(acc.at[pl.ds(_NS * _RPT, _TAIL), :],
                            out_hbm.at[pl.ds(c * _N + _NS * _RPT, _TAIL), :])

    return body(tables, init, idx)


def _build_idx(dd, dp, pp):
    c0_src = jnp.stack([dd[0], dp[1] + _N])
    c0_dst = jnp.stack([dd[1], dp[0]])
    c1_src = jnp.stack([dp[0] + 2 * _N, pp[0] + 3 * _N])
    c1_dst = jnp.stack([dp[1], pp[1]])

    ept = _E // _NS
    pad_src = jnp.tile((jnp.arange(_NPAD, dtype=jnp.int32) * 83) % (4 * _N), (_NS, 1))
    pad_dst = jnp.full((_NS, _NPAD), _N, jnp.int32) + (
        jnp.arange(_NPAD, dtype=jnp.int32) % 8)[None, :]

    def lay(a, pad):
        a = a.reshape(2, _NS, ept).transpose(1, 0, 2).reshape(_NS, _EPT)
        a = jnp.concatenate([a, pad], axis=1)
        return a.reshape(_NS, _NCHUNK, _CH)

    def core(src2, dst2):
        return jnp.stack([lay(src2, pad_src), lay(dst2, pad_dst)], axis=2)

    return jnp.stack([core(c0_src, c0_dst), core(c1_src, c1_dst)])


def kernel(h_drug, h_prot, dd_edge_index, dp_edge_index, pp_edge_index,
           W1_ds, W1_ps, W1_dd, W1_pd, W1_dp, W1_pp,
           W2_ds, W2_ps, W2_dd, W2_pd, W2_dp, W2_pp):
    h = jnp.concatenate([h_drug, h_prot], axis=0)
    idx = _build_idx(dd_edge_index, dp_edge_index, pp_edge_index)

    wmsg1 = jnp.stack([W1_dd, W1_pd, W1_dp, W1_pp])
    winit1 = jnp.stack([W1_ds, W1_ps])
    wmsg2 = jnp.stack([W2_dd, W2_pd, W2_dp, W2_pp])
    winit2 = jnp.stack([W2_ds, W2_ps])

    t1 = _mm_msg(h, wmsg1, relu_in=False)
    i1 = _mm_init(h, winit1, resid=None, relu_in=False)
    z1 = _sc_scatter(t1, i1, idx)
    t2 = _mm_msg(z1, wmsg2, relu_in=True)
    i2 = _mm_init(z1, winit2, resid=h, relu_in=True)
    z2 = _sc_scatter(t2, i2, idx)
    out = _relu_k(z2)
    return out[:_N], out[_N:]


# speedup vs baseline: 1.1333x; 1.1293x over previous
import functools

import jax
import jax.numpy as jnp
from jax import lax
from jax.experimental import pallas as pl
from jax.experimental.pallas import tpu as pltpu
from jax.experimental.pallas import tpu_sc as plsc

_N = 10000
_C = 128
_E = 160000
_NS = 16
_RPT = 624
_TAIL = _N - _NS * _RPT
_CH = 128
_NCHUNK = 160
_EPT = 2 * _E // _NS
_NPAD = _NCHUNK * _CH - _EPT
_NA = _N + 8
_BM = 1000


def _mm_layer(h, wmsg, winit, resid, relu_in):
    nb = _N // _BM

    def body(*refs):
        if resid is not None:
            h_ref, wm_ref, wi_ref, r_ref, om_ref, oi_ref = refs
        else:
            h_ref, wm_ref, wi_ref, om_ref, oi_ref = refs
        x = h_ref[...]
        if relu_in:
            x = jnp.maximum(x, 0.0)
        xb = x.astype(jnp.bfloat16)
        om_ref[0] = jnp.dot(xb, wm_ref[0, 0].astype(jnp.bfloat16),
                            preferred_element_type=jnp.float32)
        om_ref[1] = jnp.dot(xb, wm_ref[0, 1].astype(jnp.bfloat16),
                            preferred_element_type=jnp.float32)
        ini = jnp.dot(xb, wi_ref[0].astype(jnp.bfloat16),
                      preferred_element_type=jnp.float32)
        if resid is not None:
            ini = ini + r_ref[...]
        oi_ref[...] = ini

    in_specs = [
        pl.BlockSpec((_BM, _C), lambda i: (i, 0)),
        pl.BlockSpec((1, 2, _C, _C), lambda i: (i // nb, 0, 0, 0)),
        pl.BlockSpec((1, _C, _C), lambda i: (i // nb, 0, 0)),
    ]
    args = [h, wmsg, winit]
    if resid is not None:
        in_specs.append(pl.BlockSpec((_BM, _C), lambda i: (i, 0)))
        args.append(resid)
    msg, init = pl.pallas_call(
        body,
        grid=(2 * nb,),
        in_specs=in_specs,
        out_specs=[pl.BlockSpec((2, _BM, _C), lambda i: (0, i, 0)),
                   pl.BlockSpec((_BM, _C), lambda i: (i, 0))],
        out_shape=[jax.ShapeDtypeStruct((2, 2 * _N, _C), jnp.float32),
                   jax.ShapeDtypeStruct((2 * _N, _C), jnp.float32)],
    )(*args)
    return msg.reshape(4 * _N, _C), init


def _relu_k(z):
    def body(z_ref, o_ref):
        o_ref[...] = jnp.maximum(z_ref[...], 0.0)

    nb = 2 * _N // _BM
    return pl.pallas_call(
        body,
        grid=(nb,),
        in_specs=[pl.BlockSpec((_BM, _C), lambda i: (i, 0))],
        out_specs=pl.BlockSpec((_BM, _C), lambda i: (i, 0)),
        out_shape=jax.ShapeDtypeStruct((2 * _N, _C), jnp.float32),
    )(z)


def _sc_scatter(tables, init, idx):
    tables, init, idx = lax.optimization_barrier((tables, init, idx))
    mesh = plsc.VectorSubcoreMesh(core_axis_name="c", subcore_axis_name="s")

    @functools.partial(
        pl.kernel,
        out_type=jax.ShapeDtypeStruct((2 * _N, _C), jnp.float32),
        mesh=mesh,
        scratch_types=[
            pltpu.VMEM((4, 2, _CH), jnp.int32),
            pltpu.VMEM((2, _CH, _C), jnp.float32),
            pltpu.VMEM_SHARED((_NA, _C), jnp.float32),
            pltpu.SemaphoreType.DMA,
            pltpu.SemaphoreType.DMA,
            pltpu.SemaphoreType.DMA,
            pltpu.SemaphoreType.DMA,
            pltpu.SemaphoreType.DMA,
            pltpu.SemaphoreType.DMA,
            pltpu.SemaphoreType.DMA,
            pltpu.SemaphoreType.DMA,
        ],
    )
    def body(tab_hbm, init_hbm, idx_hbm, out_hbm, idx_v, rows_v, acc,
             gsem0, gsem1, ssem0, ssem1, isem0, isem1, isem2, isem3):
        gsem = (gsem0, gsem1)
        ssem = (ssem0, ssem1)
        isem = (isem0, isem1, isem2, isem3)
        c = lax.axis_index("c")
        s = lax.axis_index("s")
        r0 = c * _N + s * _RPT
        pltpu.sync_copy(init_hbm.at[pl.ds(r0, _RPT), :],
                        acc.at[pl.ds(s * _RPT, _RPT), :])

        @pl.when(s == _NS - 1)
        def _():
            pltpu.sync_copy(init_hbm.at[pl.ds(c * _N + _NS * _RPT, _TAIL), :],
                            acc.at[pl.ds(_NS * _RPT, _TAIL), :])

        plsc.subcore_barrier()

        def gather_desc(q, p):
            return pltpu.make_async_copy(tab_hbm.at[idx_v.at[q, 0]],
                                         rows_v.at[p], gsem[p])

        def scatter_desc(q, p):
            return pltpu.make_async_copy(rows_v.at[p],
                                         acc.at[idx_v.at[q, 1]], ssem[p])

        def idx_desc(j, q):
            return pltpu.make_async_copy(idx_hbm.at[c, s, j],
                                         idx_v.at[q], isem[q])

        def load_idx(j, q):
            pltpu.async_copy(idx_hbm.at[c, s, j], idx_v.at[q], isem[q])

        def scatter(q, p):
            pltpu.async_copy(rows_v.at[p], acc.at[idx_v.at[q, 1]], ssem[p],
                             add=True)

        load_idx(0, 0)
        load_idx(1, 1)
        idx_desc(0, 0).wait()
        gather_desc(0, 0).start()

        def outer(g, carry):
            for b in range(4):
                j = 4 * g + b
                p = b % 2
                gather_desc(b, p).wait()
                if b == 0:
                    @pl.when(g >= 1)
                    def _():
                        scatter_desc((b + 3) % 4, 1 - p).wait()
                else:
                    scatter_desc((b + 3) % 4, 1 - p).wait()
                if b in (0, 1, 2):
                    idx_desc(j + 1, (b + 1) % 4).wait()
                    gather_desc((b + 1) % 4, 1 - p).start()
                else:
                    @pl.when(g <= 38)
                    def _():
                        idx_desc(j + 1, 0).wait()
                        gather_desc(0, 1 - p).start()
                scatter(b, p)
                if b in (0, 1):
                    load_idx(j + 2, (b + 2) % 4)
                else:
                    @pl.when(g <= 38)
                    def _():
                        load_idx(j + 2, (b + 2) % 4)
            return carry

        lax.fori_loop(0, _NCHUNK // 4, outer, 0)
        scatter_desc(3, 1).wait()
        plsc.subcore_barrier()
        pltpu.sync_copy(acc.at[pl.ds(s * _RPT, _RPT), :],
                        out_hbm.at[pl.ds(r0, _RPT), :])

        @pl.when(s == _NS - 1)
        def _():
            pltpu.sync_copy(acc.at[pl.ds(_NS * _RPT, _TAIL), :],
                            out_hbm.at[pl.ds(c * _N + _NS * _RPT, _TAIL), :])

    return body(tables, init, idx)


def _build_idx(dd, dp, pp):
    c0_src = jnp.stack([dd[0], dp[1] + _N])
    c0_dst = jnp.stack([dd[1], dp[0]])
    c1_src = jnp.stack([dp[0] + 2 * _N, pp[0] + 3 * _N])
    c1_dst = jnp.stack([dp[1], pp[1]])

    ept = _E // _NS
    pad_src = jnp.tile((jnp.arange(_NPAD, dtype=jnp.int32) * 83) % (4 * _N), (_NS, 1))
    pad_dst = jnp.full((_NS, _NPAD), _N, jnp.int32) + (
        jnp.arange(_NPAD, dtype=jnp.int32) % 8)[None, :]

    def lay(a, pad):
        a = a.reshape(2, _NS, ept).transpose(1, 0, 2).reshape(_NS, _EPT)
        a = jnp.concatenate([a, pad], axis=1)
        return a.reshape(_NS, _NCHUNK, _CH)

    def core(src2, dst2):
        return jnp.stack([lay(src2, pad_src), lay(dst2, pad_dst)], axis=2)

    return jnp.stack([core(c0_src, c0_dst), core(c1_src, c1_dst)])


def kernel(h_drug, h_prot, dd_edge_index, dp_edge_index, pp_edge_index,
           W1_ds, W1_ps, W1_dd, W1_pd, W1_dp, W1_pp,
           W2_ds, W2_ps, W2_dd, W2_pd, W2_dp, W2_pp):
    h = jnp.concatenate([h_drug, h_prot], axis=0)
    idx = _build_idx(dd_edge_index, dp_edge_index, pp_edge_index)

    wmsg1 = jnp.stack([jnp.stack([W1_dd, W1_dp]), jnp.stack([W1_pd, W1_pp])])
    winit1 = jnp.stack([W1_ds, W1_ps])
    wmsg2 = jnp.stack([jnp.stack([W2_dd, W2_dp]), jnp.stack([W2_pd, W2_pp])])
    winit2 = jnp.stack([W2_ds, W2_ps])

    t1, i1 = _mm_layer(h, wmsg1, winit1, resid=None, relu_in=False)
    z1 = _sc_scatter(t1, i1, idx)
    t2, i2 = _mm_layer(z1, wmsg2, winit2, resid=h, relu_in=True)
    z2 = _sc_scatter(t2, i2, idx)
    out = _relu_k(z2)
    return out[:_N], out[_N:]
